# baseline (speedup 1.0000x reference)
import functools

import numpy as np

import jax
import jax.numpy as jnp
from jax import lax
from jax.experimental import pallas as pl
from jax.experimental.pallas import tpu as pltpu
from jax.experimental.pallas import tpu_sc as plsc

_NE = 10000
_NC = 2
_NS = 16
_L = 16
_C = 128


def _sc_segment_sum(x_i, row, col, val):
    half = x_i.shape[2] * 2
    e_total = val.shape[0]
    nck_total = e_total // _C
    nck = nck_total // _NS
    n_extra = nck_total - nck * _NS
    assert nck % 6 == 0

    mesh = plsc.VectorSubcoreMesh(core_axis_name="c", subcore_axis_name="s")

    @functools.partial(
        pl.kernel,
        mesh=mesh,
        compiler_params=pltpu.CompilerParams(needs_layout_passes=False, use_tc_tiling_on_sc=False),
        out_type=jax.ShapeDtypeStruct((_NC, _NE, half), jnp.float32),
        scratch_types=[
            pltpu.VMEM((_C,), jnp.int32),
            pltpu.VMEM((_C,), jnp.int32),
            pltpu.VMEM((_C,), jnp.int32),
            pltpu.VMEM((_C,), jnp.int32),
            pltpu.VMEM((_C,), jnp.int32),
            pltpu.VMEM((_C,), jnp.int32),
            pltpu.VMEM((_C,), jnp.float32),
            pltpu.VMEM((_C,), jnp.float32),
            pltpu.VMEM((_C,), jnp.float32),
            pltpu.VMEM((_C, half // 2), jnp.int32),
            pltpu.VMEM((_C, half // 2), jnp.int32),
            pltpu.VMEM((_C, half), jnp.float32),
            pltpu.VMEM((_C, half), jnp.float32),
            pltpu.VMEM_SHARED((_NE, half), jnp.float32),
            [pltpu.SemaphoreType.DMA] * 2,
            [pltpu.SemaphoreType.DMA] * 2,
            [pltpu.SemaphoreType.DMA] * 3,
        ],
    )
    def seg_sum(x_ref, row_ref, col_ref, val_ref, out_ref,
                rowb0, rowb1, rowb2, colb0, colb1, colb2,
                valb0, valb1, valb2, stb0, stb1, st0, st1,
                acc, gsem, asem, isem):
        cid = lax.axis_index("c")
        sid = lax.axis_index("s")
        ck0 = sid * nck
        rows0 = pl.multiple_of(sid * 624 + 8 * jnp.minimum(sid, 2), 8)
        has_extra = sid < n_extra

        stages_bf = [stb0, stb1]
        stages = [st0, st1]
        idxsets = [(rowb0, colb0, valb0), (rowb1, colb1, valb1),
                   (rowb2, colb2, valb2)]

        def fire_idx(ck, i3):
            rowb, colb, valb = idxsets[i3]
            off = pl.multiple_of(ck * _C, 8)
            pltpu.async_copy(row_ref.at[pl.ds(off, _C)], rowb, isem[i3])
            pltpu.async_copy(col_ref.at[pl.ds(off, _C)], colb, isem[i3])
            pltpu.async_copy(val_ref.at[pl.ds(off, _C)], valb, isem[i3])

        def wait_idx(ck, i3):
            rowb, colb, valb = idxsets[i3]
            off = pl.multiple_of(ck * _C, 8)
            pltpu.make_async_copy(row_ref.at[pl.ds(off, _C)], rowb,
                                  isem[i3]).wait()
            pltpu.make_async_copy(col_ref.at[pl.ds(off, _C)], colb,
                                  isem[i3]).wait()
            pltpu.make_async_copy(val_ref.at[pl.ds(off, _C)], valb,
                                  isem[i3]).wait()

        def fire_gather(i2, i3):
            colb = idxsets[i3][1]
            pltpu.async_copy(x_ref.at[cid].at[colb], stages_bf[i2],
                             gsem[i2])

        def wait_gather(i2, i3):
            colb = idxsets[i3][1]
            pltpu.make_async_copy(x_ref.at[cid].at[colb], stages_bf[i2],
                                  gsem[i2]).wait()

        def scale(i2, i3):
            valb = idxsets[i3][2]
            stb = stages_bf[i2]
            st = stages[i2]
            himask = jnp.int32(-65536)

            def scale_group(g, carry):
                vv16 = valb[pl.ds(g * _L, _L)]
                for l in range(_L):
                    i = g * _L + l
                    vs = vv16[l]
                    for j in range(half // (2 * _L)):
                        vi = stb[i, pl.ds(j * _L, _L)]
                        lo = plsc.bitcast(lax.shift_left(vi, 16),
                                          jnp.float32)
                        hi = plsc.bitcast(vi & himask, jnp.float32)
                        st[i, pl.ds(j * 2 * _L, _L)] = lo * vs
                        st[i, pl.ds(j * 2 * _L + _L, _L)] = hi * vs
                return carry

            lax.fori_loop(0, _C // _L, scale_group, 0)

        def fire_scatter(i2, i3):
            rowb = idxsets[i3][0]
            pltpu.async_copy(stages[i2], acc.at[rowb], asem[i2], add=True)

        def wait_scatter(i2, i3):
            rowb = idxsets[i3][0]
            pltpu.make_async_copy(stages[i2], acc.at[rowb],
                                  asem[i2]).wait()

        fire_idx(ck0, 0)
        fire_idx(ck0 + 1, 1)

        zv = jnp.zeros((_L,), jnp.float32)

        def zero_row(i, carry):
            for j in range(half // _L):
                st0[i, pl.ds(j * _L, _L)] = zv
            return carry

        lax.fori_loop(0, _C, zero_row, 0)
        for k in range(624 // _C):
            pltpu.sync_copy(st0, acc.at[pl.ds(rows0 + k * _C, _C)])
        ztail = 624 - (624 // _C) * _C
        pltpu.sync_copy(st0.at[pl.ds(0, ztail)],
                        acc.at[pl.ds(rows0 + 624 - ztail, ztail)])

        @pl.when(has_extra)
        def _():
            pltpu.sync_copy(st0.at[pl.ds(0, 8)],
                            acc.at[pl.ds(rows0 + 624, 8)])

        wait_idx(ck0, 0)
        fire_gather(0, 0)
        plsc.subcore_barrier()

        def step_body(p, carry):
            for u in range(6):
                j = p * 6 + u
                s2, s3 = u % 2, u % 3
                n2, n3 = (u + 1) % 2, (u + 1) % 3
                if u == 0:
                    @pl.when(j >= 1)
                    def _():
                        wait_scatter(n2, (u - 1) % 3)
                else:
                    wait_scatter(n2, (u - 1) % 3)
                if u in (4, 5):
                    @pl.when(p < nck // 6 - 1)
                    def _():
                        fire_idx(ck0 + j + 2, (u + 2) % 3)
                else:
                    fire_idx(ck0 + j + 2, (u + 2) % 3)
                if u == 5:
                    @pl.when(p < nck // 6 - 1)
                    def _():
                        wait_idx(ck0 + j + 1, n3)
                        fire_gather(n2, n3)
                else:
                    wait_idx(ck0 + j + 1, n3)
                    fire_gather(n2, n3)
                wait_gather(s2, s3)
                scale(s2, s3)
                fire_scatter(s2, s3)
            return carry

        lax.fori_loop(0, nck // 6, step_body, 0)
        wait_scatter((nck - 1) % 2, (nck - 1) % 3)

        @pl.when(has_extra)
        def _():
            ck = nck * _NS + sid
            fire_idx(ck, 0)
            wait_idx(ck, 0)
            fire_gather(0, 0)
            wait_gather(0, 0)
            scale(0, 0)
            fire_scatter(0, 0)
            wait_scatter(0, 0)

        plsc.subcore_barrier()
        pltpu.sync_copy(acc.at[pl.ds(rows0, 624)],
                        out_ref.at[cid, pl.ds(rows0, 624)])

        @pl.when(has_extra)
        def _():
            r1 = pl.multiple_of(rows0 + 624, 8)
            pltpu.sync_copy(acc.at[pl.ds(r1, 8)],
                            out_ref.at[cid, pl.ds(r1, 8)])

    return seg_sum(x_i, row, col, val)


def _tc_dense(x, xh2, w_node, b_node, w_edge, b_edge):
    n = x.shape[0]
    d = x.shape[1]
    h = w_node.shape[1]
    half = xh2.shape[2]
    R = 1000
    grid = (n // R,)

    def body(x_ref, xh_ref, wn_ref, bn_ref, we_ref, be_ref, on_ref, oe_ref):
        hn = jnp.dot(x_ref[...], wn_ref[...],
                     preferred_element_type=jnp.float32)
        on_ref[...] = jnp.maximum(hn + bn_ref[...], 0.0)
        we = we_ref[...]
        he = (jnp.dot(xh_ref[0], we[:half],
                      preferred_element_type=jnp.float32)
              + jnp.dot(xh_ref[1], we[half:],
                        preferred_element_type=jnp.float32))
        oe_ref[...] = jnp.maximum(he + be_ref[...], 0.0)

    return pl.pallas_call(
        body,
        grid=grid,
        in_specs=[
            pl.BlockSpec((R, d), lambda i: (i, 0)),
            pl.BlockSpec((2, R, half), lambda i: (0, i, 0)),
            pl.BlockSpec((d, h), lambda i: (0, 0)),
            pl.BlockSpec((1, h), lambda i: (0, 0)),
            pl.BlockSpec((d, h), lambda i: (0, 0)),
            pl.BlockSpec((1, h), lambda i: (0, 0)),
        ],
        out_specs=[
            pl.BlockSpec((R, h), lambda i: (i, 0)),
            pl.BlockSpec((R, h), lambda i: (i, 0)),
        ],
        out_shape=[
            jax.ShapeDtypeStruct((n, h), jnp.float32),
            jax.ShapeDtypeStruct((_NE, h), jnp.float32),
        ],
    )(x, xh2, w_node, b_node, w_edge, b_edge)


def _perm128():
    return np.concatenate(
        [np.concatenate([np.arange(32 * g, 32 * g + 32, 2),
                         np.arange(32 * g + 1, 32 * g + 32, 2)])
         for g in range(4)])


def kernel(x, incidence_indices, incidence_values, y, batch_0,
           W_node, b_node, W_edge, b_edge):
    half = x.shape[1] // 2
    row = incidence_indices[0].astype(jnp.int32)
    col = incidence_indices[1].astype(jnp.int32)
    x_bf = x.astype(jnp.bfloat16)
    xs = jnp.stack([x_bf[:, :half], x_bf[:, half:]])
    x_i = lax.bitcast_convert_type(
        xs.reshape(2, x.shape[0], half // 2, 2), jnp.int32)
    xh2 = _sc_segment_sum(x_i, row, col, incidence_values)
    p = _perm128()
    w_edge_p = jnp.concatenate([W_edge[:half][p], W_edge[half:][p]], axis=0)
    xn, xe = _tc_dense(x, xh2, W_node, b_node.reshape(1, -1),
                       w_edge_p, b_edge.reshape(1, -1))
    return (y, batch_0, xn, xe)


# speedup vs baseline: 2.0119x; 2.0119x over previous
import functools

import jax
import jax.numpy as jnp
from jax import lax
from jax.experimental import pallas as pl
from jax.experimental.pallas import tpu as pltpu
from jax.experimental.pallas import tpu_sc as plsc

_NE = 10000
_NC = 2
_NS = 16
_L = 16
_C = 128


def _sc_segment_sum(x, row, col, val):
    half = x.shape[1] // 2
    e_total = val.shape[0]
    nck_total = e_total // _C
    nck = nck_total // _NS
    n_extra = nck_total - nck * _NS
    assert nck % 6 == 0
    zr = 104

    mesh = plsc.VectorSubcoreMesh(core_axis_name="c", subcore_axis_name="s")

    @functools.partial(
        pl.kernel,
        mesh=mesh,
        out_type=jax.ShapeDtypeStruct((_NC, _NE, half), jnp.float32),
        scratch_types=[
            pltpu.VMEM((_C,), jnp.int32),
            pltpu.VMEM((_C,), jnp.int32),
            pltpu.VMEM((_C,), jnp.int32),
            pltpu.VMEM((_C,), jnp.int32),
            pltpu.VMEM((_C,), jnp.int32),
            pltpu.VMEM((_C,), jnp.int32),
            pltpu.VMEM((_C,), jnp.float32),
            pltpu.VMEM((_C,), jnp.float32),
            pltpu.VMEM((_C,), jnp.float32),
            pltpu.VMEM((_C, half), jnp.float32),
            pltpu.VMEM((_C, half), jnp.float32),
            pltpu.VMEM((zr, half), jnp.float32),
            pltpu.VMEM_SHARED((_NE, half), jnp.float32),
            [pltpu.SemaphoreType.DMA] * 2,
            [pltpu.SemaphoreType.DMA] * 2,
            [pltpu.SemaphoreType.DMA] * 3,
        ],
    )
    def seg_sum(x_ref, row_ref, col_ref, val_ref, out_ref,
                rowb0, rowb1, rowb2, colb0, colb1, colb2,
                valb0, valb1, valb2, st0, st1,
                zbuf, acc, gsem, asem, isem):
        cid = lax.axis_index("c")
        sid = lax.axis_index("s")
        col0 = pl.multiple_of(cid * half, 128)
        ck0 = sid * nck
        rows0 = pl.multiple_of(sid * 624 + 8 * jnp.minimum(sid, 2), 8)
        has_extra = sid < n_extra

        stages = [st0, st1]
        idxsets = [(rowb0, colb0, valb0), (rowb1, colb1, valb1),
                   (rowb2, colb2, valb2)]

        def fire_idx(ck, i3):
            rowb, colb, valb = idxsets[i3]
            off = pl.multiple_of(ck * _C, 8)
            pltpu.async_copy(row_ref.at[pl.ds(off, _C)], rowb, isem[i3])
            pltpu.async_copy(col_ref.at[pl.ds(off, _C)], colb, isem[i3])
            pltpu.async_copy(val_ref.at[pl.ds(off, _C)], valb, isem[i3])

        def wait_idx(ck, i3):
            rowb, colb, valb = idxsets[i3]
            off = pl.multiple_of(ck * _C, 8)
            pltpu.make_async_copy(row_ref.at[pl.ds(off, _C)], rowb,
                                  isem[i3]).wait()
            pltpu.make_async_copy(col_ref.at[pl.ds(off, _C)], colb,
                                  isem[i3]).wait()
            pltpu.make_async_copy(val_ref.at[pl.ds(off, _C)], valb,
                                  isem[i3]).wait()

        def fire_gather(i2, i3):
            colb = idxsets[i3][1]
            pltpu.async_copy(x_ref.at[colb, pl.ds(col0, half)],
                             stages[i2], gsem[i2])

        def wait_gather(i2, i3):
            colb = idxsets[i3][1]
            pltpu.make_async_copy(x_ref.at[colb, pl.ds(col0, half)],
                                  stages[i2], gsem[i2]).wait()

        def scale(i2, i3):
            valb = idxsets[i3][2]
            st = stages[i2]

            def scale_group(g, carry):
                vv16 = valb[pl.ds(g * _L, _L)]
                for l in range(_L):
                    i = g * _L + l
                    vs = vv16[l]
                    for j in range(half // _L):
                        st[i, pl.ds(j * _L, _L)] = (
                            st[i, pl.ds(j * _L, _L)] * vs)
                return carry

            lax.fori_loop(0, _C // _L, scale_group, 0)

        def fire_scatter(i2, i3):
            rowb = idxsets[i3][0]
            pltpu.async_copy(stages[i2], acc.at[rowb], asem[i2], add=True)

        def wait_scatter(i2, i3):
            rowb = idxsets[i3][0]
            pltpu.make_async_copy(stages[i2], acc.at[rowb],
                                  asem[i2]).wait()

        fire_idx(ck0, 0)
        fire_idx(ck0 + 1, 1)

        zv = jnp.zeros((_L,), jnp.float32)

        def zero_row(i, carry):
            for j in range(half // _L):
                zbuf[i, pl.ds(j * _L, _L)] = zv
            return carry

        lax.fori_loop(0, zr, zero_row, 0)
        for k in range(624 // zr):
            pltpu.sync_copy(zbuf, acc.at[pl.ds(rows0 + k * zr, zr)])

        @pl.when(has_extra)
        def _():
            pltpu.sync_copy(zbuf.at[pl.ds(0, 8)],
                            acc.at[pl.ds(rows0 + 624, 8)])

        wait_idx(ck0, 0)
        fire_gather(0, 0)
        plsc.subcore_barrier()

        def step_body(p, carry):
            for u in range(6):
                j = p * 6 + u
                s2, s3 = u % 2, u % 3
                n2, n3 = (u + 1) % 2, (u + 1) % 3
                if u == 0:
                    @pl.when(j >= 1)
                    def _():
                        wait_scatter(n2, (u - 1) % 3)
                else:
                    wait_scatter(n2, (u - 1) % 3)
                if u in (4, 5):
                    @pl.when(p < nck // 6 - 1)
                    def _():
                        fire_idx(ck0 + j + 2, (u + 2) % 3)
                else:
                    fire_idx(ck0 + j + 2, (u + 2) % 3)
                if u == 5:
                    @pl.when(p < nck // 6 - 1)
                    def _():
                        wait_idx(ck0 + j + 1, n3)
                        fire_gather(n2, n3)
                else:
                    wait_idx(ck0 + j + 1, n3)
                    fire_gather(n2, n3)
                wait_gather(s2, s3)
                scale(s2, s3)
                fire_scatter(s2, s3)
            return carry

        lax.fori_loop(0, nck // 6, step_body, 0)
        wait_scatter((nck - 1) % 2, (nck - 1) % 3)

        @pl.when(has_extra)
        def _():
            ck = nck * _NS + sid
            fire_idx(ck, 0)
            wait_idx(ck, 0)
            fire_gather(0, 0)
            wait_gather(0, 0)
            scale(0, 0)
            fire_scatter(0, 0)
            wait_scatter(0, 0)

        plsc.subcore_barrier()
        pltpu.sync_copy(acc.at[pl.ds(rows0, 624)],
                        out_ref.at[cid, pl.ds(rows0, 624)])

        @pl.when(has_extra)
        def _():
            r1 = pl.multiple_of(rows0 + 624, 8)
            pltpu.sync_copy(acc.at[pl.ds(r1, 8)],
                            out_ref.at[cid, pl.ds(r1, 8)])

    return seg_sum(x, row, col, val)


def _tc_dense(x, xh2, w_node, b_node, w_edge, b_edge):
    n = x.shape[0]
    d = x.shape[1]
    h = w_node.shape[1]
    half = xh2.shape[2]
    R = 1000
    grid = (n // R,)

    def body(x_ref, xh_ref, wn_ref, bn_ref, we_ref, be_ref, on_ref, oe_ref):
        hn = jnp.dot(x_ref[...], wn_ref[...],
                     preferred_element_type=jnp.float32)
        on_ref[...] = jnp.maximum(hn + bn_ref[...], 0.0)
        we = we_ref[...]
        he = (jnp.dot(xh_ref[0], we[:half],
                      preferred_element_type=jnp.float32)
              + jnp.dot(xh_ref[1], we[half:],
                        preferred_element_type=jnp.float32))
        oe_ref[...] = jnp.maximum(he + be_ref[...], 0.0)

    return pl.pallas_call(
        body,
        grid=grid,
        in_specs=[
            pl.BlockSpec((R, d), lambda i: (i, 0)),
            pl.BlockSpec((2, R, half), lambda i: (0, i, 0)),
            pl.BlockSpec((d, h), lambda i: (0, 0)),
            pl.BlockSpec((1, h), lambda i: (0, 0)),
            pl.BlockSpec((d, h), lambda i: (0, 0)),
            pl.BlockSpec((1, h), lambda i: (0, 0)),
        ],
        out_specs=[
            pl.BlockSpec((R, h), lambda i: (i, 0)),
            pl.BlockSpec((R, h), lambda i: (i, 0)),
        ],
        out_shape=[
            jax.ShapeDtypeStruct((n, h), jnp.float32),
            jax.ShapeDtypeStruct((_NE, h), jnp.float32),
        ],
    )(x, xh2, w_node, b_node, w_edge, b_edge)


def kernel(x, incidence_indices, incidence_values, y, batch_0,
           W_node, b_node, W_edge, b_edge):
    row = incidence_indices[0].astype(jnp.int32)
    col = incidence_indices[1].astype(jnp.int32)
    xh2 = _sc_segment_sum(x, row, col, incidence_values)
    xn, xe = _tc_dense(x, xh2, W_node, b_node.reshape(1, -1),
                       W_edge, b_edge.reshape(1, -1))
    return (y, batch_0, xn, xe)
